# Initial kernel scaffold; baseline (speedup 1.0000x reference)
import jax
import jax.numpy as jnp
from jax.experimental import pallas as pl


def kernel(x, rbf, i, j, tag, emb_table, tag_table, W_rbf, b_rbf, W_lin, b_lin):
    raise NotImplementedError("write your pallas kernel here")


# speedup vs baseline: 2.3292x; 2.3292x over previous
import functools

import jax
import jax.numpy as jnp
from jax import lax
from jax.experimental import pallas as pl
from jax.experimental.pallas import tpu as pltpu
from jax.experimental.pallas import tpu_sc as plsc


def _node_proj_kernel(x_ref, tag_ref, emb_ref, tagt_ref, w1_ref, w2_ref,
                      a_ref, b_ref):
    n = x_ref.shape[0]
    n_emb_rows = emb_ref.shape[0]
    n_tag_rows = tagt_ref.shape[0]
    xv = x_ref[...]
    tv = tag_ref[...]
    ohx = (xv[:, None] == lax.broadcasted_iota(jnp.int32, (n, n_emb_rows), 1)
           ).astype(jnp.float32)
    oht = (tv[:, None] == lax.broadcasted_iota(jnp.int32, (n, n_tag_rows), 1)
           ).astype(jnp.float32)
    xe = jnp.dot(ohx, emb_ref[...], preferred_element_type=jnp.float32)
    xt = jnp.dot(oht, tagt_ref[...], preferred_element_type=jnp.float32)
    xf = jnp.concatenate([xe, xt], axis=1)
    a_ref[...] = jnp.dot(xf, w1_ref[...], preferred_element_type=jnp.float32)
    b_ref[...] = jnp.dot(xf, w2_ref[...], preferred_element_type=jnp.float32)


def _node_proj(x, tag, emb_table, tag_table, w1, w2):
    n = x.shape[0]
    h = w1.shape[1]
    return pl.pallas_call(
        _node_proj_kernel,
        out_shape=(jax.ShapeDtypeStruct((n, h), jnp.float32),
                   jax.ShapeDtypeStruct((n, h), jnp.float32)),
    )(x, tag, emb_table, tag_table, w1, w2)


def _sc_gather(a, b, idx_i, idx_j):
    n, h = a.shape
    e = idx_i.shape[0]
    info = plsc.get_sparse_core_info()
    nw = info.num_cores * info.num_subcores
    per_w = e // nw
    chunk = 80
    n_chunks = per_w // chunk
    assert per_w * nw == e and n_chunks * chunk == per_w

    mesh = plsc.VectorSubcoreMesh(core_axis_name="c", subcore_axis_name="s")

    @functools.partial(
        pl.kernel,
        mesh=mesh,
        out_type=[jax.ShapeDtypeStruct((e, h), jnp.float32),
                  jax.ShapeDtypeStruct((e, h), jnp.float32)],
        scratch_types=[
            pltpu.VMEM((chunk,), jnp.int32),
            pltpu.VMEM((chunk,), jnp.int32),
            pltpu.VMEM((chunk, h), jnp.float32),
            pltpu.VMEM((chunk, h), jnp.float32),
            pltpu.SemaphoreType.DMA,
            pltpu.SemaphoreType.DMA,
        ],
    )
    def gather_kernel(a_hbm, b_hbm, i_hbm, j_hbm, ga_hbm, gb_hbm,
                      ivec, jvec, rows_a, rows_b, sem_a, sem_b):
        wid = lax.axis_index("s") * info.num_cores + lax.axis_index("c")
        base = wid * per_w

        def body(c, carry):
            off = base + c * chunk
            pltpu.sync_copy(i_hbm.at[pl.ds(off, chunk)], ivec)
            pltpu.sync_copy(j_hbm.at[pl.ds(off, chunk)], jvec)
            cp_a = pltpu.async_copy(a_hbm.at[ivec], rows_a, sem_a)
            cp_b = pltpu.async_copy(b_hbm.at[jvec], rows_b, sem_b)
            cp_a.wait()
            pltpu.sync_copy(rows_a, ga_hbm.at[pl.ds(off, chunk)])
            cp_b.wait()
            pltpu.sync_copy(rows_b, gb_hbm.at[pl.ds(off, chunk)])
            return carry

        lax.fori_loop(0, n_chunks, body, 0)

    return gather_kernel(a, b, idx_i, idx_j)


def _combine_kernel(rbf_ref, ga_ref, gb_ref, wr_ref, br_ref, w3_ref, bl_ref,
                    out_ref):
    rb = rbf_ref[...]
    h1 = jnp.dot(rb, wr_ref[...], preferred_element_type=jnp.float32) + br_ref[...]
    h1 = h1 * jax.nn.sigmoid(h1)
    c = (jnp.dot(h1, w3_ref[...], preferred_element_type=jnp.float32)
         + bl_ref[...] + ga_ref[...] + gb_ref[...])
    out_ref[...] = c * jax.nn.sigmoid(c)


def _combine(rbf, ga, gb, w_rbf, b_rbf, w3, b_lin, block_e=2000):
    e, h = ga.shape
    nrad = rbf.shape[1]
    grid = (e // block_e,)
    return pl.pallas_call(
        _combine_kernel,
        grid=grid,
        in_specs=[
            pl.BlockSpec((block_e, nrad), lambda i: (i, 0)),
            pl.BlockSpec((block_e, h), lambda i: (i, 0)),
            pl.BlockSpec((block_e, h), lambda i: (i, 0)),
            pl.BlockSpec((nrad, h), lambda i: (0, 0)),
            pl.BlockSpec((1, h), lambda i: (0, 0)),
            pl.BlockSpec((h, h), lambda i: (0, 0)),
            pl.BlockSpec((1, h), lambda i: (0, 0)),
        ],
        out_specs=pl.BlockSpec((block_e, h), lambda i: (i, 0)),
        out_shape=jax.ShapeDtypeStruct((e, h), jnp.float32),
    )(rbf, ga, gb, w_rbf, b_rbf, w3, b_lin)


def kernel(x, rbf, i, j, tag, emb_table, tag_table, W_rbf, b_rbf, W_lin, b_lin):
    h = W_lin.shape[1]
    w1 = W_lin[0:h]
    w2 = W_lin[h:2 * h]
    w3 = W_lin[2 * h:3 * h]
    a, b = _node_proj(x.astype(jnp.int32), tag.astype(jnp.int32),
                      emb_table, tag_table, w1, w2)
    ga, gb = _sc_gather(a, b, i.astype(jnp.int32), j.astype(jnp.int32))
    return _combine(rbf, ga, gb, W_rbf, b_rbf.reshape(1, h), w3,
                    b_lin.reshape(1, h))


# speedup vs baseline: 5.2017x; 2.2333x over previous
import functools

import jax
import jax.numpy as jnp
from jax import lax
from jax.experimental import pallas as pl
from jax.experimental.pallas import tpu as pltpu
from jax.experimental.pallas import tpu_sc as plsc


def _node_proj_kernel(x_ref, tag_ref, emb_ref, tagt_ref, w1_ref, w2_ref,
                      a_ref, b_ref):
    n = x_ref.shape[0]
    n_emb_rows = emb_ref.shape[0]
    n_tag_rows = tagt_ref.shape[0]
    xv = x_ref[...]
    tv = tag_ref[...]
    ohx = (xv[:, None] == lax.broadcasted_iota(jnp.int32, (n, n_emb_rows), 1)
           ).astype(jnp.float32)
    oht = (tv[:, None] == lax.broadcasted_iota(jnp.int32, (n, n_tag_rows), 1)
           ).astype(jnp.float32)
    xe = jnp.dot(ohx, emb_ref[...], preferred_element_type=jnp.float32)
    xt = jnp.dot(oht, tagt_ref[...], preferred_element_type=jnp.float32)
    xf = jnp.concatenate([xe, xt], axis=1)
    a_ref[...] = jnp.dot(xf, w1_ref[...], preferred_element_type=jnp.float32)
    b_ref[...] = jnp.dot(xf, w2_ref[...], preferred_element_type=jnp.float32)


def _node_proj(x, tag, emb_table, tag_table, w1, w2):
    n = x.shape[0]
    h = w1.shape[1]
    return pl.pallas_call(
        _node_proj_kernel,
        out_shape=(jax.ShapeDtypeStruct((n, h), jnp.float32),
                   jax.ShapeDtypeStruct((n, h), jnp.float32)),
    )(x, tag, emb_table, tag_table, w1, w2)


def _sc_gather(a, b, idx_i, idx_j):
    n, h = a.shape
    dt = a.dtype
    e = idx_i.shape[0]
    info = plsc.get_sparse_core_info()
    nw = info.num_cores * info.num_subcores
    per_w = e // nw
    nbuf = 5
    chunk = 80
    n_chunks = per_w // chunk
    n_rounds = n_chunks // nbuf
    assert per_w * nw == e and n_chunks * chunk == per_w and n_chunks % nbuf == 0

    mesh = plsc.VectorSubcoreMesh(core_axis_name="c", subcore_axis_name="s")

    @functools.partial(
        pl.kernel,
        mesh=mesh,
        out_type=jax.ShapeDtypeStruct((e // 2, h), dt),
        scratch_types=[
            pltpu.VMEM((per_w,), jnp.int32),
            pltpu.VMEM((per_w,), jnp.int32),
            pltpu.VMEM((nbuf, chunk, h), dt),
            pltpu.VMEM((nbuf, chunk, h), dt),
        ] + [pltpu.SemaphoreType.DMA] * (3 * nbuf),
    )
    def gather_kernel(a_hbm, b_hbm, i_hbm, j_hbm, g_hbm,
                      ivec, jvec, rows_a, rows_b, *sems):
        wid = lax.axis_index("s") * info.num_cores + lax.axis_index("c")
        woff = wid * per_w
        sga = sems[:nbuf]
        sgb = sems[nbuf:2 * nbuf]
        ssa = sems[2 * nbuf:3 * nbuf]

        pltpu.sync_copy(i_hbm.at[pl.ds(woff, per_w)], ivec)
        pltpu.sync_copy(j_hbm.at[pl.ds(woff, per_w)], jvec)

        def gather_descs(g, s):
            sl = pl.ds(g * chunk, chunk)
            return (pltpu.make_async_copy(a_hbm.at[ivec.at[sl]], rows_a.at[s],
                                          sga[s]),
                    pltpu.make_async_copy(b_hbm.at[jvec.at[sl]], rows_b.at[s],
                                          sgb[s]))

        half = chunk // 2
        woff2 = wid * (per_w // 2)

        def scatter_desc(g, s):
            sl = pl.ds(woff2 + g * half, half)
            return pltpu.make_async_copy(rows_a.at[s, pl.ds(0, half)],
                                         g_hbm.at[sl], ssa[s])

        def issue_gather(g, s):
            for d in gather_descs(g, s):
                d.start()

        def wait_gather(g, s):
            for d in gather_descs(g, s):
                d.wait()

        def add_rows(s):
            rnd = jnp.uint32(0x8000)

            def rowbody(r, carry):
                for cc in range(h // 16):
                    sl = pl.ds(cc * 16, 16)
                    s_lo = rows_a[s, r, sl] + rows_b[s, r, sl]
                    s_hi = rows_a[s, r + half, sl] + rows_b[s, r + half, sl]
                    u_lo = lax.bitcast_convert_type(s_lo, jnp.uint32)
                    u_hi = lax.bitcast_convert_type(s_hi, jnp.uint32)
                    word = (((u_lo + rnd) >> 16)
                            | ((u_hi + rnd) & jnp.uint32(0xFFFF0000)))
                    rows_a[s, r, sl] = lax.bitcast_convert_type(
                        word, jnp.float32)
                return carry
            lax.fori_loop(0, half, rowbody, 0)

        for s in range(nbuf):
            issue_gather(s, s)

        def body(it, carry):
            g0 = it * nbuf
            for s in range(nbuf):
                wait_gather(g0 + s, s)
                add_rows(s)
                scatter_desc(g0 + s, s).start()
            for s in range(nbuf):
                scatter_desc(g0 + s, s).wait()

                @pl.when(it < n_rounds - 1)
                def _():
                    issue_gather(g0 + nbuf + s, s)
            return carry

        lax.fori_loop(0, n_rounds, body, 0)

    return gather_kernel(a, b, idx_i, idx_j)


_PAIR = 40


def _combine_kernel(rbf_ref, g_ref, wr_ref, br_ref, w3_ref, bl_ref,
                    out_ref):
    rbt = rbf_ref[...]
    h1 = lax.dot_general(rbt, wr_ref[...], (((0,), (0,)), ((), ())),
                         preferred_element_type=jnp.float32) + br_ref[...]
    h1 = h1 * jax.nn.sigmoid(h1)
    xi = lax.bitcast_convert_type(g_ref[...], jnp.int32)
    lo = lax.bitcast_convert_type(xi << 16, jnp.float32)
    hi = lax.bitcast_convert_type(xi & jnp.int32(-65536), jnp.float32)
    nch = xi.shape[0] // _PAIR
    h = xi.shape[1]
    gf = jnp.concatenate(
        [lo.reshape(nch, _PAIR, h), hi.reshape(nch, _PAIR, h)], axis=1
    ).reshape(2 * xi.shape[0], h)
    c = (jnp.dot(h1, w3_ref[...], preferred_element_type=jnp.float32)
         + bl_ref[...] + gf)
    out_ref[...] = c * jax.nn.sigmoid(c)


def _combine_part_kernel(prev_ref, rbf_ref, g_ref, wr_ref, br_ref, w3_ref,
                         bl_ref, out_ref):
    _combine_kernel(rbf_ref, g_ref, wr_ref, br_ref, w3_ref, bl_ref, out_ref)


def _combine_part(rbf, g, w_rbf, b_rbf, w3, b_lin, part, prev, block_e=2560):
    h = g.shape[1]
    ep = g.shape[0] * 2
    nrad, e = rbf.shape
    nblk = ep // block_e
    base = part * nblk
    specs = [
        pl.BlockSpec((nrad, block_e), lambda k: (0, base + k)),
        pl.BlockSpec((block_e // 2, h), lambda k: (k, 0)),
        pl.BlockSpec((nrad, h), lambda k: (0, 0)),
        pl.BlockSpec((1, h), lambda k: (0, 0)),
        pl.BlockSpec((h, h), lambda k: (0, 0)),
        pl.BlockSpec((1, h), lambda k: (0, 0)),
    ]
    out_spec = pl.BlockSpec((block_e, h), lambda k: (base + k, 0))
    out_shape = jax.ShapeDtypeStruct((e, h), jnp.float32)
    if prev is None:
        return pl.pallas_call(
            _combine_kernel,
            grid=(nblk,),
            in_specs=specs,
            out_specs=out_spec,
            out_shape=out_shape,
        )(rbf, g, w_rbf, b_rbf, w3, b_lin)
    return pl.pallas_call(
        _combine_part_kernel,
        grid=(nblk,),
        in_specs=[pl.BlockSpec((8, h), lambda k: (0, 0))] + specs,
        out_specs=out_spec,
        out_shape=out_shape,
        input_output_aliases={0: 0},
    )(prev, rbf, g, w_rbf, b_rbf, w3, b_lin)


def kernel(x, rbf, i, j, tag, emb_table, tag_table, W_rbf, b_rbf, W_lin, b_lin):
    h = W_lin.shape[1]
    w1 = W_lin[0:h]
    w2 = W_lin[h:2 * h]
    w3 = W_lin[2 * h:3 * h]
    a, b = _node_proj(x.astype(jnp.int32), tag.astype(jnp.int32),
                      emb_table, tag_table, w1, w2)
    i32 = i.astype(jnp.int32)
    j32 = j.astype(jnp.int32)
    br = b_rbf.reshape(1, h)
    bl = b_lin.reshape(1, h)
    e = i32.shape[0]
    rbf_t = rbf.T
    parts = 5
    ep = e // parts
    out = None
    for p in range(parts):
        gp = _sc_gather(a, b,
                        lax.slice(i32, (p * ep,), ((p + 1) * ep,)),
                        lax.slice(j32, (p * ep,), ((p + 1) * ep,)))
        out = _combine_part(rbf_t, gp, W_rbf, br, w3, bl, p, out)
    return out


# speedup vs baseline: 5.2327x; 1.0060x over previous
import functools

import jax
import jax.numpy as jnp
from jax import lax
from jax.experimental import pallas as pl
from jax.experimental.pallas import tpu as pltpu
from jax.experimental.pallas import tpu_sc as plsc


def _node_proj_kernel(x_ref, tag_ref, emb_ref, tagt_ref, w1_ref, w2_ref,
                      a_ref, b_ref):
    n = x_ref.shape[0]
    n_emb_rows = emb_ref.shape[0]
    n_tag_rows = tagt_ref.shape[0]
    xv = x_ref[...]
    tv = tag_ref[...]
    ohx = (xv[:, None] == lax.broadcasted_iota(jnp.int32, (n, n_emb_rows), 1)
           ).astype(jnp.float32)
    oht = (tv[:, None] == lax.broadcasted_iota(jnp.int32, (n, n_tag_rows), 1)
           ).astype(jnp.float32)
    xe = jnp.dot(ohx, emb_ref[...], preferred_element_type=jnp.float32)
    xt = jnp.dot(oht, tagt_ref[...], preferred_element_type=jnp.float32)
    xf = jnp.concatenate([xe, xt], axis=1)
    a_ref[...] = jnp.dot(xf, w1_ref[...], preferred_element_type=jnp.float32)
    b_ref[...] = jnp.dot(xf, w2_ref[...], preferred_element_type=jnp.float32)


def _node_proj(x, tag, emb_table, tag_table, w1, w2):
    n = x.shape[0]
    h = w1.shape[1]
    return pl.pallas_call(
        _node_proj_kernel,
        out_shape=(jax.ShapeDtypeStruct((n, h), jnp.float32),
                   jax.ShapeDtypeStruct((n, h), jnp.float32)),
    )(x, tag, emb_table, tag_table, w1, w2)


def _sc_gather(a, b, idx_i, idx_j, part, ep):
    n, h = a.shape
    dt = a.dtype
    e = ep
    ebase = part * ep
    info = plsc.get_sparse_core_info()
    nw = info.num_cores * info.num_subcores
    per_w = e // nw
    nbuf = 5
    chunk = 80
    n_chunks = per_w // chunk
    n_rounds = n_chunks // nbuf
    assert per_w * nw == e and n_chunks * chunk == per_w and n_chunks % nbuf == 0

    mesh = plsc.VectorSubcoreMesh(core_axis_name="c", subcore_axis_name="s")

    @functools.partial(
        pl.kernel,
        mesh=mesh,
        out_type=jax.ShapeDtypeStruct((e // 2, h), dt),
        scratch_types=[
            pltpu.VMEM((per_w,), jnp.int32),
            pltpu.VMEM((per_w,), jnp.int32),
            pltpu.VMEM((nbuf, chunk, h), dt),
            pltpu.VMEM((nbuf, chunk, h), dt),
        ] + [pltpu.SemaphoreType.DMA] * (3 * nbuf),
    )
    def gather_kernel(a_hbm, b_hbm, i_hbm, j_hbm, g_hbm,
                      ivec, jvec, rows_a, rows_b, *sems):
        wid = lax.axis_index("s") * info.num_cores + lax.axis_index("c")
        woff = wid * per_w
        sga = sems[:nbuf]
        sgb = sems[nbuf:2 * nbuf]
        ssa = sems[2 * nbuf:3 * nbuf]

        pltpu.sync_copy(i_hbm.at[pl.ds(ebase + woff, per_w)], ivec)
        pltpu.sync_copy(j_hbm.at[pl.ds(ebase + woff, per_w)], jvec)

        def gather_descs(g, s):
            sl = pl.ds(g * chunk, chunk)
            return (pltpu.make_async_copy(a_hbm.at[ivec.at[sl]], rows_a.at[s],
                                          sga[s]),
                    pltpu.make_async_copy(b_hbm.at[jvec.at[sl]], rows_b.at[s],
                                          sgb[s]))

        half = chunk // 2
        woff2 = wid * (per_w // 2)

        def scatter_desc(g, s):
            sl = pl.ds(woff2 + g * half, half)
            return pltpu.make_async_copy(rows_a.at[s, pl.ds(0, half)],
                                         g_hbm.at[sl], ssa[s])

        def issue_gather(g, s):
            for d in gather_descs(g, s):
                d.start()

        def wait_gather(g, s):
            for d in gather_descs(g, s):
                d.wait()

        def add_rows(s):
            rnd = jnp.uint32(0x8000)

            def rowbody(r, carry):
                for cc in range(h // 16):
                    sl = pl.ds(cc * 16, 16)
                    s_lo = rows_a[s, r, sl] + rows_b[s, r, sl]
                    s_hi = rows_a[s, r + half, sl] + rows_b[s, r + half, sl]
                    u_lo = lax.bitcast_convert_type(s_lo, jnp.uint32)
                    u_hi = lax.bitcast_convert_type(s_hi, jnp.uint32)
                    word = (((u_lo + rnd) >> 16)
                            | ((u_hi + rnd) & jnp.uint32(0xFFFF0000)))
                    rows_a[s, r, sl] = lax.bitcast_convert_type(
                        word, jnp.float32)
                return carry
            lax.fori_loop(0, half, rowbody, 0)

        for s in range(nbuf):
            issue_gather(s, s)

        def body(it, carry):
            g0 = it * nbuf
            for s in range(nbuf):
                wait_gather(g0 + s, s)
                add_rows(s)
                scatter_desc(g0 + s, s).start()
            for s in range(nbuf):
                scatter_desc(g0 + s, s).wait()

                @pl.when(it < n_rounds - 1)
                def _():
                    issue_gather(g0 + nbuf + s, s)
            return carry

        lax.fori_loop(0, n_rounds, body, 0)

    return gather_kernel(a, b, idx_i, idx_j)


_PAIR = 40


def _combine_kernel(rbf_ref, g_ref, wr_ref, br_ref, w3_ref, bl_ref,
                    out_ref):
    rbt = rbf_ref[...]
    h1 = lax.dot_general(rbt, wr_ref[...], (((0,), (0,)), ((), ())),
                         preferred_element_type=jnp.float32) + br_ref[...]
    h1 = h1 * jax.nn.sigmoid(h1)
    xi = lax.bitcast_convert_type(g_ref[...], jnp.int32)
    lo = lax.bitcast_convert_type(xi << 16, jnp.float32)
    hi = lax.bitcast_convert_type(xi & jnp.int32(-65536), jnp.float32)
    nch = xi.shape[0] // _PAIR
    h = xi.shape[1]
    gf = jnp.concatenate(
        [lo.reshape(nch, _PAIR, h), hi.reshape(nch, _PAIR, h)], axis=1
    ).reshape(2 * xi.shape[0], h)
    c = (jnp.dot(h1, w3_ref[...], preferred_element_type=jnp.float32)
         + bl_ref[...] + gf)
    out_ref[...] = c * jax.nn.sigmoid(c)


def _combine_part_kernel(prev_ref, rbf_ref, g_ref, wr_ref, br_ref, w3_ref,
                         bl_ref, out_ref):
    _combine_kernel(rbf_ref, g_ref, wr_ref, br_ref, w3_ref, bl_ref, out_ref)


def _combine_part(rbf, g, w_rbf, b_rbf, w3, b_lin, part, prev, block_e=2560):
    h = g.shape[1]
    ep = g.shape[0] * 2
    nrad, e = rbf.shape
    nblk = ep // block_e
    base = part * nblk
    specs = [
        pl.BlockSpec((nrad, block_e), lambda k: (0, base + k)),
        pl.BlockSpec((block_e // 2, h), lambda k: (k, 0)),
        pl.BlockSpec((nrad, h), lambda k: (0, 0)),
        pl.BlockSpec((1, h), lambda k: (0, 0)),
        pl.BlockSpec((h, h), lambda k: (0, 0)),
        pl.BlockSpec((1, h), lambda k: (0, 0)),
    ]
    out_spec = pl.BlockSpec((block_e, h), lambda k: (base + k, 0))
    out_shape = jax.ShapeDtypeStruct((e, h), jnp.float32)
    if prev is None:
        return pl.pallas_call(
            _combine_kernel,
            grid=(nblk,),
            in_specs=specs,
            out_specs=out_spec,
            out_shape=out_shape,
        )(rbf, g, w_rbf, b_rbf, w3, b_lin)
    return pl.pallas_call(
        _combine_part_kernel,
        grid=(nblk,),
        in_specs=[pl.BlockSpec((8, h), lambda k: (0, 0))] + specs,
        out_specs=out_spec,
        out_shape=out_shape,
        input_output_aliases={0: 0},
    )(prev, rbf, g, w_rbf, b_rbf, w3, b_lin)


def kernel(x, rbf, i, j, tag, emb_table, tag_table, W_rbf, b_rbf, W_lin, b_lin):
    h = W_lin.shape[1]
    w1 = W_lin[0:h]
    w2 = W_lin[h:2 * h]
    w3 = W_lin[2 * h:3 * h]
    a, b = _node_proj(x.astype(jnp.int32), tag.astype(jnp.int32),
                      emb_table, tag_table, w1, w2)
    i32 = i.astype(jnp.int32)
    j32 = j.astype(jnp.int32)
    br = b_rbf.reshape(1, h)
    bl = b_lin.reshape(1, h)
    e = i32.shape[0]
    rbf_t = rbf.T
    parts = 5
    ep = e // parts
    out = None
    for p in range(parts):
        gp = _sc_gather(a, b, i32, j32, p, ep)
        out = _combine_part(rbf_t, gp, W_rbf, br, w3, bl, p, out)
    return out


# speedup vs baseline: 5.2513x; 1.0035x over previous
import functools

import jax
import jax.numpy as jnp
from jax import lax
from jax.experimental import pallas as pl
from jax.experimental.pallas import tpu as pltpu
from jax.experimental.pallas import tpu_sc as plsc


def _node_proj_kernel(x_ref, tag_ref, emb_ref, tagt_ref, w1_ref, w2_ref,
                      a_ref, b_ref):
    n = x_ref.shape[0]
    n_emb_rows = emb_ref.shape[0]
    n_tag_rows = tagt_ref.shape[0]
    xv = x_ref[...]
    tv = tag_ref[...]
    ohx = (xv[:, None] == lax.broadcasted_iota(jnp.int32, (n, n_emb_rows), 1)
           ).astype(jnp.float32)
    oht = (tv[:, None] == lax.broadcasted_iota(jnp.int32, (n, n_tag_rows), 1)
           ).astype(jnp.float32)
    xe = jnp.dot(ohx, emb_ref[...], preferred_element_type=jnp.float32)
    xt = jnp.dot(oht, tagt_ref[...], preferred_element_type=jnp.float32)
    xf = jnp.concatenate([xe, xt], axis=1)
    a_ref[...] = jnp.dot(xf, w1_ref[...], preferred_element_type=jnp.float32)
    b_ref[...] = jnp.dot(xf, w2_ref[...], preferred_element_type=jnp.float32)


def _node_proj(x, tag, emb_table, tag_table, w1, w2):
    n = x.shape[0]
    h = w1.shape[1]
    return pl.pallas_call(
        _node_proj_kernel,
        out_shape=(jax.ShapeDtypeStruct((n, h), jnp.float32),
                   jax.ShapeDtypeStruct((n, h), jnp.float32)),
    )(x, tag, emb_table, tag_table, w1, w2)


def _sc_gather(a, b, idx_i, idx_j, ebase, ep):
    n, h = a.shape
    dt = a.dtype
    e = ep
    info = plsc.get_sparse_core_info()
    nw = info.num_cores * info.num_subcores
    per_w = e // nw
    nbuf = 5
    chunk = 80
    n_chunks = per_w // chunk
    n_rounds = n_chunks // nbuf
    assert per_w * nw == e and n_chunks * chunk == per_w and n_chunks % nbuf == 0

    mesh = plsc.VectorSubcoreMesh(core_axis_name="c", subcore_axis_name="s")

    @functools.partial(
        pl.kernel,
        mesh=mesh,
        out_type=jax.ShapeDtypeStruct((e // 2, h), dt),
        scratch_types=[
            pltpu.VMEM((per_w,), jnp.int32),
            pltpu.VMEM((per_w,), jnp.int32),
            pltpu.VMEM((nbuf, chunk, h), dt),
            pltpu.VMEM((nbuf, chunk, h), dt),
        ] + [pltpu.SemaphoreType.DMA] * (3 * nbuf),
    )
    def gather_kernel(a_hbm, b_hbm, i_hbm, j_hbm, g_hbm,
                      ivec, jvec, rows_a, rows_b, *sems):
        wid = lax.axis_index("s") * info.num_cores + lax.axis_index("c")
        woff = wid * per_w
        sga = sems[:nbuf]
        sgb = sems[nbuf:2 * nbuf]
        ssa = sems[2 * nbuf:3 * nbuf]

        pltpu.sync_copy(i_hbm.at[pl.ds(ebase + woff, per_w)], ivec)
        pltpu.sync_copy(j_hbm.at[pl.ds(ebase + woff, per_w)], jvec)

        def gather_descs(g, s):
            sl = pl.ds(g * chunk, chunk)
            return (pltpu.make_async_copy(a_hbm.at[ivec.at[sl]], rows_a.at[s],
                                          sga[s]),
                    pltpu.make_async_copy(b_hbm.at[jvec.at[sl]], rows_b.at[s],
                                          sgb[s]))

        half = chunk // 2
        woff2 = wid * (per_w // 2)

        def scatter_desc(g, s):
            sl = pl.ds(woff2 + g * half, half)
            return pltpu.make_async_copy(rows_a.at[s, pl.ds(0, half)],
                                         g_hbm.at[sl], ssa[s])

        def issue_gather(g, s):
            for d in gather_descs(g, s):
                d.start()

        def wait_gather(g, s):
            for d in gather_descs(g, s):
                d.wait()

        def add_rows(s):
            rnd = jnp.uint32(0x8000)

            def rowbody(r, carry):
                for cc in range(h // 16):
                    sl = pl.ds(cc * 16, 16)
                    s_lo = rows_a[s, r, sl] + rows_b[s, r, sl]
                    s_hi = rows_a[s, r + half, sl] + rows_b[s, r + half, sl]
                    u_lo = lax.bitcast_convert_type(s_lo, jnp.uint32)
                    u_hi = lax.bitcast_convert_type(s_hi, jnp.uint32)
                    word = (((u_lo + rnd) >> 16)
                            | ((u_hi + rnd) & jnp.uint32(0xFFFF0000)))
                    rows_a[s, r, sl] = lax.bitcast_convert_type(
                        word, jnp.float32)
                return carry
            lax.fori_loop(0, half, rowbody, 0)

        for s in range(nbuf):
            issue_gather(s, s)

        def body(it, carry):
            g0 = it * nbuf
            for s in range(nbuf):
                wait_gather(g0 + s, s)
                add_rows(s)
                scatter_desc(g0 + s, s).start()
            for s in range(nbuf):
                scatter_desc(g0 + s, s).wait()

                @pl.when(it < n_rounds - 1)
                def _():
                    issue_gather(g0 + nbuf + s, s)
            return carry

        lax.fori_loop(0, n_rounds, body, 0)

    return gather_kernel(a, b, idx_i, idx_j)


_PAIR = 40


def _combine_kernel(rbf_ref, g_ref, wr_ref, br_ref, w3_ref, bl_ref,
                    out_ref):
    rbt = rbf_ref[...]
    h1 = lax.dot_general(rbt, wr_ref[...], (((0,), (0,)), ((), ())),
                         preferred_element_type=jnp.float32) + br_ref[...]
    h1 = h1 * jax.nn.sigmoid(h1)
    xi = lax.bitcast_convert_type(g_ref[...], jnp.int32)
    lo = lax.bitcast_convert_type(xi << 16, jnp.float32)
    hi = lax.bitcast_convert_type(xi & jnp.int32(-65536), jnp.float32)
    nch = xi.shape[0] // _PAIR
    h = xi.shape[1]
    gf = jnp.concatenate(
        [lo.reshape(nch, _PAIR, h), hi.reshape(nch, _PAIR, h)], axis=1
    ).reshape(2 * xi.shape[0], h)
    c = (jnp.dot(h1, w3_ref[...], preferred_element_type=jnp.float32)
         + bl_ref[...] + gf)
    out_ref[...] = c * jax.nn.sigmoid(c)


def _combine_part_kernel(prev_ref, rbf_ref, g_ref, wr_ref, br_ref, w3_ref,
                         bl_ref, out_ref):
    _combine_kernel(rbf_ref, g_ref, wr_ref, br_ref, w3_ref, bl_ref, out_ref)


def _combine_part(rbf, g, w_rbf, b_rbf, w3, b_lin, base, prev, block_e=2560):
    h = g.shape[1]
    ep = g.shape[0] * 2
    nrad, e = rbf.shape
    nblk = ep // block_e
    specs = [
        pl.BlockSpec((nrad, block_e), lambda k: (0, base + k)),
        pl.BlockSpec((block_e // 2, h), lambda k: (k, 0)),
        pl.BlockSpec((nrad, h), lambda k: (0, 0)),
        pl.BlockSpec((1, h), lambda k: (0, 0)),
        pl.BlockSpec((h, h), lambda k: (0, 0)),
        pl.BlockSpec((1, h), lambda k: (0, 0)),
    ]
    out_spec = pl.BlockSpec((block_e, h), lambda k: (base + k, 0))
    out_shape = jax.ShapeDtypeStruct((e, h), jnp.float32)
    if prev is None:
        return pl.pallas_call(
            _combine_kernel,
            grid=(nblk,),
            in_specs=specs,
            out_specs=out_spec,
            out_shape=out_shape,
        )(rbf, g, w_rbf, b_rbf, w3, b_lin)
    return pl.pallas_call(
        _combine_part_kernel,
        grid=(nblk,),
        in_specs=[pl.BlockSpec((8, h), lambda k: (0, 0))] + specs,
        out_specs=out_spec,
        out_shape=out_shape,
        input_output_aliases={0: 0},
    )(prev, rbf, g, w_rbf, b_rbf, w3, b_lin)


def kernel(x, rbf, i, j, tag, emb_table, tag_table, W_rbf, b_rbf, W_lin, b_lin):
    h = W_lin.shape[1]
    w1 = W_lin[0:h]
    w2 = W_lin[h:2 * h]
    w3 = W_lin[2 * h:3 * h]
    a, b = _node_proj(x.astype(jnp.int32), tag.astype(jnp.int32),
                      emb_table, tag_table, w1, w2)
    i32 = i.astype(jnp.int32)
    j32 = j.astype(jnp.int32)
    br = b_rbf.reshape(1, h)
    bl = b_lin.reshape(1, h)
    e = i32.shape[0]
    rbf_t = rbf.T
    unit = e // 25
    block_e = 2560
    out = None
    ebase = 0
    for units in (1, 6, 6, 6, 6):
        ep = units * unit
        gp = _sc_gather(a, b, i32, j32, ebase, ep)
        out = _combine_part(rbf_t, gp, W_rbf, br, w3, bl,
                            ebase // block_e, out, block_e=block_e)
        ebase += ep
    return out
